# baseline (speedup 1.0000x reference)
import functools

import jax
import jax.numpy as jnp
from jax import lax
from jax.experimental import pallas as pl
from jax.experimental.pallas import tpu as pltpu
from jax.experimental.pallas import tpu_sc as plsc

_IOU_T = 0.5
_MAX_OUT = 1000
_B = 512
_OUT_PAD = 1024


def _iou_gt_tile(r, c):
    rx1, ry1, rx2, ry2 = r
    cx1, cy1, cx2, cy2 = c
    area_r = (rx2 - rx1) * (ry2 - ry1)
    area_c = (cx2 - cx1) * (cy2 - cy1)
    ix1 = jnp.maximum(rx1, cx1)
    iy1 = jnp.maximum(ry1, cy1)
    ix2 = jnp.minimum(rx2, cx2)
    iy2 = jnp.minimum(ry2, cy2)
    iw = jnp.maximum(ix2 - ix1, 0.0)
    ih = jnp.maximum(iy2 - iy1, 0.0)
    inter = iw * ih
    union = area_r + area_c - inter + 1e-9
    iou = inter / union
    return (iou > _IOU_T).astype(jnp.float32)


def _row_coords(d_ref, start):
    return tuple(d_ref[pl.ds(start, _B), i:i + 1] for i in range(4))


def _rowdot(v, m):
    return lax.dot_general(v, m, (((1,), (0,)), ((), ())),
                           preferred_element_type=jnp.float32)


def _transpose_row(x_col, ident):
    return lax.dot_general(x_col, ident, (((0,), (0,)), ((), ())),
                           precision=lax.Precision.HIGHEST,
                           preferred_element_type=jnp.float32)


def _rank_body(nblocks, s_col_ref, s_row_ref, rank_ref):
    f32 = jnp.float32
    ri = lax.broadcasted_iota(jnp.int32, (_B, _B), 0)
    ci = lax.broadcasted_iota(jnp.int32, (_B, _B), 1)
    tri = ri < ci
    ones = jnp.ones((1, _B), f32)
    for c in range(nblocks):
        sc = s_row_ref[0:1, c * _B:(c + 1) * _B]
        acc = jnp.zeros((1, _B), f32)
        for r in range(nblocks):
            sr = s_col_ref[pl.ds(r * _B, _B), 0:1]
            if r < c:
                cmp = sr >= sc
            elif r > c:
                cmp = sr > sc
            else:
                cmp = (sr > sc) | ((sr == sc) & tri)
            acc = acc + _rowdot(ones, cmp.astype(f32))
        rank_ref[0:1, c * _B:(c + 1) * _B] = acc.astype(jnp.int32)


def _nms_body(n_valid, nblocks, d_ref, dest_ref, keep_ref):
    f32 = jnp.float32
    ri = lax.broadcasted_iota(jnp.int32, (_B, _B), 0)
    ci = lax.broadcasted_iota(jnp.int32, (_B, _B), 1)
    tri = (ri < ci).astype(f32)
    cum = (ri <= ci).astype(f32)
    ident = (ri == ci).astype(f32)

    for k in range(nblocks):
        cb = k * _B
        rows_self = _row_coords(d_ref, cb)
        cols = tuple(_transpose_row(x, ident) for x in rows_self)

        def _cross(r, acc, cols=cols):
            rows = _row_coords(d_ref, r * _B)
            m = _iou_gt_tile(rows, cols)
            kr = keep_ref[pl.ds(r, 1), :]
            return acc + _rowdot(kr, m)

        supp_cnt = jnp.zeros((1, _B), f32)
        if k > 0:
            supp_cnt = lax.fori_loop(0, k, _cross, supp_cnt)
        notsupp = (supp_cnt == 0.0).astype(f32)

        m_self = _iou_gt_tile(rows_self, cols) * tri

        def _fp_cond(st):
            return st[1]

        def _fp_body(st, m_self=m_self, notsupp=notsupp):
            kf = st[0]
            cnt = _rowdot(kf * notsupp, m_self)
            kf2 = (cnt == 0.0).astype(f32)
            return kf2, jnp.any(kf2 != kf)

        kf, _ = lax.while_loop(_fp_cond, _fp_body,
                               (jnp.ones((1, _B), f32), jnp.bool_(True)))
        keep_ref[pl.ds(k, 1), :] = kf * notsupp

    lane = lax.broadcasted_iota(jnp.int32, (1, _B), 1)
    kept_chunks, ck_chunks, cs_chunks, valid_chunks = [], [], [], []
    ck_carry = jnp.zeros((), f32)
    cs_carry = jnp.zeros((), f32)
    for k in range(nblocks):
        valid = ((lane + k * _B) < n_valid).astype(f32)
        kp = keep_ref[pl.ds(k, 1), :]
        kv = kp * valid
        sv = (1.0 - kp) * valid
        ck = _rowdot(kv, cum) + ck_carry
        cs = _rowdot(sv, cum) + cs_carry
        ck_carry = ck[0, _B - 1]
        cs_carry = cs[0, _B - 1]
        kept_chunks.append(kv)
        ck_chunks.append(ck)
        cs_chunks.append(cs)
        valid_chunks.append(valid)
    total_kept = ck_carry

    for k in range(nblocks):
        dest = jnp.where(kept_chunks[k] > 0.0,
                         ck_chunks[k] - 1.0,
                         total_kept + cs_chunks[k] - 1.0).astype(jnp.int32)
        ok = (valid_chunks[k] > 0.0) & (dest < _MAX_OUT)
        dump = _OUT_PAD + lax.rem(lane, jnp.int32(128))
        dest_ref[0:1, k * _B:(k + 1) * _B] = jnp.where(ok, dest, dump)


def _sc_scatter_rows(table, dest2d, out_rows):
    npad, width = table.shape
    info = plsc.get_sparse_core_info()
    ncores = info.num_cores
    nw = ncores * info.num_subcores
    bpw = npad // nw
    mesh = plsc.VectorSubcoreMesh(core_axis_name="c", subcore_axis_name="s")

    @functools.partial(
        pl.kernel, mesh=mesh,
        compiler_params=pltpu.CompilerParams(use_tc_tiling_on_sc=False),
        out_type=jax.ShapeDtypeStruct((out_rows, width), jnp.float32),
        scratch_types=[
            pltpu.VMEM((bpw,), jnp.int32),
            pltpu.VMEM((bpw, width), jnp.float32),
            pltpu.SemaphoreType.DMA,
        ],
    )
    def k(table_hbm, dest_hbm, out_hbm, idx_v, rows_v, sem):
        wid = lax.axis_index("s") * ncores + lax.axis_index("c")
        pltpu.sync_copy(dest_hbm.at[wid], idx_v)
        pltpu.sync_copy(table_hbm.at[pl.ds(wid * bpw, bpw)], rows_v)
        pltpu.async_copy(rows_v, out_hbm.at[idx_v], sem).wait()

    return k(table, dest2d)


def kernel(boxes, scores):
    n = boxes.shape[0]
    nblocks = -(-n // _B)
    npad = nblocks * _B
    nw = 32
    s = scores.astype(jnp.float32)
    s_col = jnp.full((npad, 1), -jnp.inf, jnp.float32).at[:n, 0].set(s)
    s_row = s_col.reshape(1, npad)
    d16 = (jnp.zeros((npad, 16), jnp.float32)
           .at[:n, :4].set(boxes.astype(jnp.float32))
           .at[:n, 4].set(s))

    rank = pl.pallas_call(
        functools.partial(_rank_body, nblocks),
        out_shape=jax.ShapeDtypeStruct((1, npad), jnp.int32),
    )(s_col, s_row)
    d16s = _sc_scatter_rows(d16, rank.reshape(nw, npad // nw), npad)

    dest = pl.pallas_call(
        functools.partial(_nms_body, n, nblocks),
        out_shape=jax.ShapeDtypeStruct((1, npad), jnp.int32),
        scratch_shapes=[pltpu.VMEM((max(8, nblocks), _B), jnp.float32)],
    )(d16s)
    out16 = _sc_scatter_rows(d16s, dest.reshape(nw, npad // nw),
                             _OUT_PAD + 128)
    return out16[:_MAX_OUT, :5]


# speedup vs baseline: 1.0840x; 1.0840x over previous
import functools

import jax
import jax.numpy as jnp
from jax import lax
from jax.experimental import pallas as pl
from jax.experimental.pallas import tpu as pltpu
from jax.experimental.pallas import tpu_sc as plsc

_IOU_T = 0.5
_MAX_OUT = 1000
_B = 512
_OUT_PAD = 1024


def _iou_gt_tile(r, c):
    rx1, ry1, rx2, ry2 = r
    cx1, cy1, cx2, cy2 = c
    area_r = (rx2 - rx1) * (ry2 - ry1)
    area_c = (cx2 - cx1) * (cy2 - cy1)
    ix1 = jnp.maximum(rx1, cx1)
    iy1 = jnp.maximum(ry1, cy1)
    ix2 = jnp.minimum(rx2, cx2)
    iy2 = jnp.minimum(ry2, cy2)
    iw = jnp.maximum(ix2 - ix1, 0.0)
    ih = jnp.maximum(iy2 - iy1, 0.0)
    inter = iw * ih
    union = area_r + area_c - inter + 1e-9
    iou = inter / union
    return (iou > _IOU_T).astype(jnp.float32)


def _row_coords(d_ref, start):
    return tuple(d_ref[pl.ds(start, _B), i:i + 1] for i in range(4))


def _rowdot(v, m):
    return lax.dot_general(v, m, (((1,), (0,)), ((), ())),
                           preferred_element_type=jnp.float32)


def _transpose_row(x_col, ident):
    return lax.dot_general(x_col, ident, (((0,), (0,)), ((), ())),
                           precision=lax.Precision.HIGHEST,
                           preferred_element_type=jnp.float32)


def _rank_body(nblocks, s_col_ref, s_row_ref, rank_ref):
    f32 = jnp.float32
    ri = lax.broadcasted_iota(jnp.int32, (_B, _B), 0)
    ci = lax.broadcasted_iota(jnp.int32, (_B, _B), 1)
    tri = ri < ci
    ones = jnp.ones((1, _B), f32)
    for c in range(nblocks):
        sc = s_row_ref[0:1, c * _B:(c + 1) * _B]
        acc = jnp.zeros((1, _B), f32)
        for r in range(nblocks):
            sr = s_col_ref[pl.ds(r * _B, _B), 0:1]
            if r < c:
                cmp = sr >= sc
            elif r > c:
                cmp = sr > sc
            else:
                cmp = (sr > sc) | ((sr == sc) & tri)
            acc = acc + _rowdot(ones, cmp.astype(f32))
        rank_ref[0:1, c * _B:(c + 1) * _B] = acc.astype(jnp.int32)


def _nms_body(n_valid, nblocks, bT_ref, d_ref, dest_ref, keep_ref):
    f32 = jnp.float32
    ri = lax.broadcasted_iota(jnp.int32, (_B, _B), 0)
    ci = lax.broadcasted_iota(jnp.int32, (_B, _B), 1)
    tri = (ri < ci).astype(f32)
    cum = (ri <= ci).astype(f32)

    for k in range(nblocks):
        cb = k * _B
        rows_self = _row_coords(d_ref, cb)
        cols = tuple(bT_ref[i:i + 1, cb:cb + _B] for i in range(4))

        def _cross(r, acc, cols=cols):
            rows = _row_coords(d_ref, r * _B)
            m = _iou_gt_tile(rows, cols)
            kr = keep_ref[pl.ds(r, 1), :]
            return acc + _rowdot(kr, m)

        supp_cnt = jnp.zeros((1, _B), f32)
        if k > 0:
            supp_cnt = lax.fori_loop(0, k, _cross, supp_cnt)
        notsupp = (supp_cnt == 0.0).astype(f32)

        m_self = _iou_gt_tile(rows_self, cols) * tri

        def _fp_cond(st):
            return st[1]

        def _fp_body(st, m_self=m_self, notsupp=notsupp):
            kf = st[0]
            cnt = _rowdot(kf * notsupp, m_self)
            kf2 = (cnt == 0.0).astype(f32)
            return kf2, jnp.any(kf2 != kf)

        kf, _ = lax.while_loop(_fp_cond, _fp_body,
                               (jnp.ones((1, _B), f32), jnp.bool_(True)))
        keep_ref[pl.ds(k, 1), :] = kf * notsupp

    lane = lax.broadcasted_iota(jnp.int32, (1, _B), 1)
    kept_chunks, ck_chunks, cs_chunks, valid_chunks = [], [], [], []
    ck_carry = jnp.zeros((), f32)
    cs_carry = jnp.zeros((), f32)
    for k in range(nblocks):
        valid = ((lane + k * _B) < n_valid).astype(f32)
        kp = keep_ref[pl.ds(k, 1), :]
        kv = kp * valid
        sv = (1.0 - kp) * valid
        ck = _rowdot(kv, cum) + ck_carry
        cs = _rowdot(sv, cum) + cs_carry
        ck_carry = ck[0, _B - 1]
        cs_carry = cs[0, _B - 1]
        kept_chunks.append(kv)
        ck_chunks.append(ck)
        cs_chunks.append(cs)
        valid_chunks.append(valid)
    total_kept = ck_carry

    for k in range(nblocks):
        dest = jnp.where(kept_chunks[k] > 0.0,
                         ck_chunks[k] - 1.0,
                         total_kept + cs_chunks[k] - 1.0).astype(jnp.int32)
        ok = (valid_chunks[k] > 0.0) & (dest < _MAX_OUT)
        dump = _OUT_PAD + lax.rem(lane, jnp.int32(128))
        dest_ref[0:1, k * _B:(k + 1) * _B] = jnp.where(ok, dest, dump)


def _sc_scatter_rows(table, dest2d, out_rows):
    npad, width = table.shape
    info = plsc.get_sparse_core_info()
    ncores = info.num_cores
    nw = ncores * info.num_subcores
    bpw = npad // nw
    mesh = plsc.VectorSubcoreMesh(core_axis_name="c", subcore_axis_name="s")

    @functools.partial(
        pl.kernel, mesh=mesh,
        compiler_params=pltpu.CompilerParams(use_tc_tiling_on_sc=False),
        out_type=jax.ShapeDtypeStruct((out_rows, width), jnp.float32),
        scratch_types=[
            pltpu.VMEM((bpw,), jnp.int32),
            pltpu.VMEM((bpw, width), jnp.float32),
            pltpu.SemaphoreType.DMA,
        ],
    )
    def k(table_hbm, dest_hbm, out_hbm, idx_v, rows_v, sem):
        wid = lax.axis_index("s") * ncores + lax.axis_index("c")
        pltpu.sync_copy(dest_hbm.at[wid], idx_v)
        pltpu.sync_copy(table_hbm.at[pl.ds(wid * bpw, bpw)], rows_v)
        pltpu.async_copy(rows_v, out_hbm.at[idx_v], sem).wait()

    return k(table, dest2d)


def kernel(boxes, scores):
    n = boxes.shape[0]
    nblocks = -(-n // _B)
    npad = nblocks * _B
    nw = 32
    s = scores.astype(jnp.float32)
    s_col = jnp.full((npad, 1), -jnp.inf, jnp.float32).at[:n, 0].set(s)
    s_row = s_col.reshape(1, npad)
    d16 = (jnp.zeros((npad, 16), jnp.float32)
           .at[:n, :4].set(boxes.astype(jnp.float32))
           .at[:n, 4].set(s))

    rank = pl.pallas_call(
        functools.partial(_rank_body, nblocks),
        out_shape=jax.ShapeDtypeStruct((1, npad), jnp.int32),
    )(s_col, s_row)
    d16s = _sc_scatter_rows(d16, rank.reshape(nw, npad // nw), npad)
    bT = d16s[:, :4].T

    dest = pl.pallas_call(
        functools.partial(_nms_body, n, nblocks),
        out_shape=jax.ShapeDtypeStruct((1, npad), jnp.int32),
        scratch_shapes=[pltpu.VMEM((max(8, nblocks), _B), jnp.float32)],
    )(bT, d16s)
    out16 = _sc_scatter_rows(d16s, dest.reshape(nw, npad // nw),
                             _OUT_PAD + 128)
    return out16[:_MAX_OUT, :5]


# speedup vs baseline: 1.0926x; 1.0079x over previous
import functools

import jax
import jax.numpy as jnp
from jax import lax
from jax.experimental import pallas as pl
from jax.experimental.pallas import tpu as pltpu
from jax.experimental.pallas import tpu_sc as plsc

_IOU_T = 0.5
_MAX_OUT = 1000
_B = 512
_OUT_PAD = 1024


def _iou_gt_tile(r, c):
    rx1, ry1, rx2, ry2 = r
    cx1, cy1, cx2, cy2 = c
    area_r = (rx2 - rx1) * (ry2 - ry1)
    area_c = (cx2 - cx1) * (cy2 - cy1)
    ix1 = jnp.maximum(rx1, cx1)
    iy1 = jnp.maximum(ry1, cy1)
    ix2 = jnp.minimum(rx2, cx2)
    iy2 = jnp.minimum(ry2, cy2)
    iw = jnp.maximum(ix2 - ix1, 0.0)
    ih = jnp.maximum(iy2 - iy1, 0.0)
    inter = iw * ih
    union = area_r + area_c - inter + 1e-9
    iou = inter / union
    return (iou > _IOU_T).astype(jnp.float32)


def _row_coords(d_ref, start):
    return tuple(d_ref[pl.ds(start, _B), i:i + 1] for i in range(4))


def _rowdot(v, m):
    return lax.dot_general(v, m, (((1,), (0,)), ((), ())),
                           preferred_element_type=jnp.float32)


def _transpose_row(x_col, ident):
    return lax.dot_general(x_col, ident, (((0,), (0,)), ((), ())),
                           precision=lax.Precision.HIGHEST,
                           preferred_element_type=jnp.float32)


def _rank_body(nblocks, s_col_ref, s_row_ref, rank_ref):
    f32 = jnp.float32
    ri = lax.broadcasted_iota(jnp.int32, (_B, _B), 0)
    ci = lax.broadcasted_iota(jnp.int32, (_B, _B), 1)
    tri = ri < ci
    ones = jnp.ones((1, _B), f32)
    for c in range(nblocks):
        sc = s_row_ref[0:1, c * _B:(c + 1) * _B]
        acc = jnp.zeros((1, _B), f32)
        for r in range(nblocks):
            sr = s_col_ref[pl.ds(r * _B, _B), 0:1]
            if r < c:
                cmp = sr >= sc
            elif r > c:
                cmp = sr > sc
            else:
                cmp = (sr > sc) | ((sr == sc) & tri)
            acc = acc + _rowdot(ones, cmp.astype(f32))
        rank_ref[0:1, c * _B:(c + 1) * _B] = acc.astype(jnp.int32)


def _nms_body(n_valid, nblocks, bT_ref, d_ref, dest_ref, keep_ref):
    f32 = jnp.float32
    ri = lax.broadcasted_iota(jnp.int32, (_B, _B), 0)
    ci = lax.broadcasted_iota(jnp.int32, (_B, _B), 1)
    tri = (ri < ci).astype(f32)
    cum = (ri <= ci).astype(f32)

    for k in range(nblocks):
        cb = k * _B
        rows_self = _row_coords(d_ref, cb)
        cols = tuple(bT_ref[i:i + 1, cb:cb + _B] for i in range(4))

        def _cross(r, acc, cols=cols):
            rows = _row_coords(d_ref, r * _B)
            m = _iou_gt_tile(rows, cols)
            kr = keep_ref[pl.ds(r, 1), :]
            return acc + _rowdot(kr, m)

        supp_cnt = jnp.zeros((1, _B), f32)
        if k > 0:
            supp_cnt = lax.fori_loop(0, k, _cross, supp_cnt)
        notsupp = (supp_cnt == 0.0).astype(f32)

        m_self = _iou_gt_tile(rows_self, cols) * tri

        def _fp_cond(st):
            return st[1]

        def _fp_body(st, m_self=m_self, notsupp=notsupp):
            kf = st[0]
            kf1 = (_rowdot(kf * notsupp, m_self) == 0.0).astype(f32)
            kf2 = (_rowdot(kf1 * notsupp, m_self) == 0.0).astype(f32)
            return kf2, jnp.any(kf2 != kf1)

        kf, _ = lax.while_loop(_fp_cond, _fp_body,
                               (jnp.ones((1, _B), f32), jnp.bool_(True)))
        keep_ref[pl.ds(k, 1), :] = kf * notsupp

    lane = lax.broadcasted_iota(jnp.int32, (1, _B), 1)
    kept_chunks, ck_chunks, cs_chunks, valid_chunks = [], [], [], []
    ck_carry = jnp.zeros((), f32)
    cs_carry = jnp.zeros((), f32)
    for k in range(nblocks):
        valid = ((lane + k * _B) < n_valid).astype(f32)
        kp = keep_ref[pl.ds(k, 1), :]
        kv = kp * valid
        sv = (1.0 - kp) * valid
        ck = _rowdot(kv, cum) + ck_carry
        cs = _rowdot(sv, cum) + cs_carry
        ck_carry = ck[0, _B - 1]
        cs_carry = cs[0, _B - 1]
        kept_chunks.append(kv)
        ck_chunks.append(ck)
        cs_chunks.append(cs)
        valid_chunks.append(valid)
    total_kept = ck_carry

    for k in range(nblocks):
        dest = jnp.where(kept_chunks[k] > 0.0,
                         ck_chunks[k] - 1.0,
                         total_kept + cs_chunks[k] - 1.0).astype(jnp.int32)
        ok = (valid_chunks[k] > 0.0) & (dest < _MAX_OUT)
        dump = _OUT_PAD + lax.rem(lane, jnp.int32(128))
        dest_ref[0:1, k * _B:(k + 1) * _B] = jnp.where(ok, dest, dump)


def _sc_scatter_rows(table, dest2d, out_rows):
    npad, width = table.shape
    info = plsc.get_sparse_core_info()
    ncores = info.num_cores
    nw = ncores * info.num_subcores
    bpw = npad // nw
    mesh = plsc.VectorSubcoreMesh(core_axis_name="c", subcore_axis_name="s")

    @functools.partial(
        pl.kernel, mesh=mesh,
        compiler_params=pltpu.CompilerParams(use_tc_tiling_on_sc=False),
        out_type=jax.ShapeDtypeStruct((out_rows, width), jnp.float32),
        scratch_types=[
            pltpu.VMEM((bpw,), jnp.int32),
            pltpu.VMEM((bpw, width), jnp.float32),
            pltpu.SemaphoreType.DMA,
        ],
    )
    def k(table_hbm, dest_hbm, out_hbm, idx_v, rows_v, sem):
        wid = lax.axis_index("s") * ncores + lax.axis_index("c")
        pltpu.sync_copy(dest_hbm.at[wid], idx_v)
        pltpu.sync_copy(table_hbm.at[pl.ds(wid * bpw, bpw)], rows_v)
        pltpu.async_copy(rows_v, out_hbm.at[idx_v], sem).wait()

    return k(table, dest2d)


def kernel(boxes, scores):
    n = boxes.shape[0]
    nblocks = -(-n // _B)
    npad = nblocks * _B
    nw = 32
    s = scores.astype(jnp.float32)
    s_col = jnp.full((npad, 1), -jnp.inf, jnp.float32).at[:n, 0].set(s)
    s_row = s_col.reshape(1, npad)
    d16 = (jnp.zeros((npad, 16), jnp.float32)
           .at[:n, :4].set(boxes.astype(jnp.float32))
           .at[:n, 4].set(s))

    rank = pl.pallas_call(
        functools.partial(_rank_body, nblocks),
        out_shape=jax.ShapeDtypeStruct((1, npad), jnp.int32),
    )(s_col, s_row)
    d16s = _sc_scatter_rows(d16, rank.reshape(nw, npad // nw), npad)
    bT = d16s[:, :4].T

    dest = pl.pallas_call(
        functools.partial(_nms_body, n, nblocks),
        out_shape=jax.ShapeDtypeStruct((1, npad), jnp.int32),
        scratch_shapes=[pltpu.VMEM((max(8, nblocks), _B), jnp.float32)],
    )(bT, d16s)
    out16 = _sc_scatter_rows(d16s, dest.reshape(nw, npad // nw),
                             _OUT_PAD + 128)
    return out16[:_MAX_OUT, :5]


# speedup vs baseline: 1.2348x; 1.1302x over previous
import functools

import jax
import jax.numpy as jnp
from jax import lax
from jax.experimental import pallas as pl
from jax.experimental.pallas import tpu as pltpu
from jax.experimental.pallas import tpu_sc as plsc

_IOU_T = 0.5
_MAX_OUT = 1000
_B = 512
_OUT_PAD = 1024


def _iou_gt_tile(r, c):
    rx1, ry1, rx2, ry2 = r
    cx1, cy1, cx2, cy2 = c
    area_r = (rx2 - rx1) * (ry2 - ry1)
    area_c = (cx2 - cx1) * (cy2 - cy1)
    ix1 = jnp.maximum(rx1, cx1)
    iy1 = jnp.maximum(ry1, cy1)
    ix2 = jnp.minimum(rx2, cx2)
    iy2 = jnp.minimum(ry2, cy2)
    iw = jnp.maximum(ix2 - ix1, 0.0)
    ih = jnp.maximum(iy2 - iy1, 0.0)
    inter = iw * ih
    union = area_r + area_c - inter + 1e-9
    iou = inter / union
    return (iou > _IOU_T).astype(jnp.float32)


def _row_coords(d_ref, start):
    return tuple(d_ref[pl.ds(start, _B), i:i + 1] for i in range(4))


def _rowdot(v, m):
    return lax.dot_general(v, m, (((1,), (0,)), ((), ())),
                           preferred_element_type=jnp.float32)


def _transpose_row(x_col, ident):
    return lax.dot_general(x_col, ident, (((0,), (0,)), ((), ())),
                           precision=lax.Precision.HIGHEST,
                           preferred_element_type=jnp.float32)


def _rank_body(nblocks, bp_ref, s_col_ref, s_row_ref, rank_ref, d16_ref):
    f32 = jnp.float32
    npad = nblocks * _B
    d16_ref[...] = jnp.concatenate(
        [bp_ref[...], s_col_ref[...], jnp.zeros((npad, 11), f32)], axis=1)
    ri = lax.broadcasted_iota(jnp.int32, (_B, _B), 0)
    ci = lax.broadcasted_iota(jnp.int32, (_B, _B), 1)
    tri = ri < ci
    ones = jnp.ones((1, _B), f32)
    for c in range(nblocks):
        sc = s_row_ref[0:1, c * _B:(c + 1) * _B]
        acc = jnp.zeros((1, _B), f32)
        for r in range(nblocks):
            sr = s_col_ref[pl.ds(r * _B, _B), 0:1]
            if r < c:
                cmp = sr >= sc
            elif r > c:
                cmp = sr > sc
            else:
                cmp = (sr > sc) | ((sr == sc) & tri)
            acc = acc + _rowdot(ones, cmp.astype(f32))
        rank_ref[0:1, c * _B:(c + 1) * _B] = acc.astype(jnp.int32)


def _nms_body(n_valid, nblocks, bT_ref, d_ref, dest_ref, keep_ref):
    f32 = jnp.float32
    ri = lax.broadcasted_iota(jnp.int32, (_B, _B), 0)
    ci = lax.broadcasted_iota(jnp.int32, (_B, _B), 1)
    tri = (ri < ci).astype(f32)
    cum = (ri <= ci).astype(f32)

    for k in range(nblocks):
        cb = k * _B
        rows_self = _row_coords(d_ref, cb)
        cols = tuple(bT_ref[i:i + 1, cb:cb + _B] for i in range(4))

        def _cross(r, acc, cols=cols):
            rows = _row_coords(d_ref, r * _B)
            m = _iou_gt_tile(rows, cols)
            kr = keep_ref[pl.ds(r, 1), :]
            return acc + _rowdot(kr, m)

        supp_cnt = jnp.zeros((1, _B), f32)
        if k > 0:
            supp_cnt = lax.fori_loop(0, k, _cross, supp_cnt)
        notsupp = (supp_cnt == 0.0).astype(f32)

        m_self = _iou_gt_tile(rows_self, cols) * tri

        def _fp_cond(st):
            return st[1]

        def _fp_body(st, m_self=m_self, notsupp=notsupp):
            kf = st[0]
            kf1 = (_rowdot(kf * notsupp, m_self) == 0.0).astype(f32)
            kf2 = (_rowdot(kf1 * notsupp, m_self) == 0.0).astype(f32)
            return kf2, jnp.any(kf2 != kf1)

        kf, _ = lax.while_loop(_fp_cond, _fp_body,
                               (jnp.ones((1, _B), f32), jnp.bool_(True)))
        keep_ref[pl.ds(k, 1), :] = kf * notsupp

    lane = lax.broadcasted_iota(jnp.int32, (1, _B), 1)
    kept_chunks, ck_chunks, cs_chunks, valid_chunks = [], [], [], []
    ck_carry = jnp.zeros((), f32)
    cs_carry = jnp.zeros((), f32)
    for k in range(nblocks):
        valid = ((lane + k * _B) < n_valid).astype(f32)
        kp = keep_ref[pl.ds(k, 1), :]
        kv = kp * valid
        sv = (1.0 - kp) * valid
        ck = _rowdot(kv, cum) + ck_carry
        cs = _rowdot(sv, cum) + cs_carry
        ck_carry = ck[0, _B - 1]
        cs_carry = cs[0, _B - 1]
        kept_chunks.append(kv)
        ck_chunks.append(ck)
        cs_chunks.append(cs)
        valid_chunks.append(valid)
    total_kept = ck_carry

    for k in range(nblocks):
        dest = jnp.where(kept_chunks[k] > 0.0,
                         ck_chunks[k] - 1.0,
                         total_kept + cs_chunks[k] - 1.0).astype(jnp.int32)
        ok = (valid_chunks[k] > 0.0) & (dest < _MAX_OUT)
        dump = _OUT_PAD + lax.rem(lane, jnp.int32(128))
        dest_ref[0:1, k * _B:(k + 1) * _B] = jnp.where(ok, dest, dump)


def _sc_scatter_rows(table, dest2d, out_rows):
    npad, width = table.shape
    info = plsc.get_sparse_core_info()
    ncores = info.num_cores
    nw = ncores * info.num_subcores
    bpw = npad // nw
    mesh = plsc.VectorSubcoreMesh(core_axis_name="c", subcore_axis_name="s")

    @functools.partial(
        pl.kernel, mesh=mesh,
        compiler_params=pltpu.CompilerParams(use_tc_tiling_on_sc=False),
        out_type=jax.ShapeDtypeStruct((out_rows, width), jnp.float32),
        scratch_types=[
            pltpu.VMEM((bpw,), jnp.int32),
            pltpu.VMEM((bpw, width), jnp.float32),
            pltpu.SemaphoreType.DMA,
        ],
    )
    def k(table_hbm, dest_hbm, out_hbm, idx_v, rows_v, sem):
        wid = lax.axis_index("s") * ncores + lax.axis_index("c")
        pltpu.sync_copy(dest_hbm.at[wid], idx_v)
        pltpu.sync_copy(table_hbm.at[pl.ds(wid * bpw, bpw)], rows_v)
        pltpu.async_copy(rows_v, out_hbm.at[idx_v], sem).wait()

    return k(table, dest2d)


def kernel(boxes, scores):
    n = boxes.shape[0]
    nblocks = -(-n // _B)
    npad = nblocks * _B
    nw = 32
    s = scores.astype(jnp.float32)
    s_col = jnp.full((npad, 1), -jnp.inf, jnp.float32).at[:n, 0].set(s)
    s_row = s_col.reshape(1, npad)
    bp = jnp.pad(boxes.astype(jnp.float32), ((0, npad - n), (0, 0)))

    rank, d16 = pl.pallas_call(
        functools.partial(_rank_body, nblocks),
        out_shape=[jax.ShapeDtypeStruct((1, npad), jnp.int32),
                   jax.ShapeDtypeStruct((npad, 16), jnp.float32)],
    )(bp, s_col, s_row)
    d16s = _sc_scatter_rows(d16, rank.reshape(nw, npad // nw), npad)
    bT = d16s[:, :4].T

    dest = pl.pallas_call(
        functools.partial(_nms_body, n, nblocks),
        out_shape=jax.ShapeDtypeStruct((1, npad), jnp.int32),
        scratch_shapes=[pltpu.VMEM((max(8, nblocks), _B), jnp.float32)],
    )(bT, d16s)
    out16 = _sc_scatter_rows(d16s, dest.reshape(nw, npad // nw),
                             _OUT_PAD + 128)
    return out16[:_MAX_OUT, :5]
